# Initial kernel scaffold; baseline (speedup 1.0000x reference)
import jax
import jax.numpy as jnp
from jax.experimental import pallas as pl


def kernel(x, weight):
    raise NotImplementedError("write your pallas kernel here")


# speedup vs baseline: 5.7838x; 5.7838x over previous
import functools

import jax
import jax.numpy as jnp
from jax import lax
from jax.experimental import pallas as pl
from jax.experimental.pallas import tpu as pltpu
from jax.experimental.pallas import tpu_sc as plsc

CHUNK = 128


@functools.lru_cache(maxsize=None)
def _make_gather(V, D, B):
    info = plsc.get_sparse_core_info()
    NC, NS = info.num_cores, info.num_subcores
    NW = NC * NS
    assert B % (NW * CHUNK) == 0
    n_chunks = B // (NW * CHUNK)
    mesh = plsc.VectorSubcoreMesh(core_axis_name="c", subcore_axis_name="s")

    @functools.partial(
        pl.kernel,
        mesh=mesh,
        out_type=jax.ShapeDtypeStruct((B, D), jnp.float32),
        scratch_types=[
            pltpu.VMEM((n_chunks, CHUNK), jnp.int32),
            pltpu.VMEM((CHUNK, D), jnp.float32),
            pltpu.SemaphoreType.DMA,
        ],
    )
    def grab(x_hbm, w_hbm, out_hbm, idx_v, rows_v, sem):
        wid = lax.axis_index("s") * NC + lax.axis_index("c")
        pltpu.sync_copy(x_hbm.at[wid], idx_v)

        def body(j, carry):
            pltpu.async_copy(w_hbm.at[idx_v.at[j]], rows_v, sem).wait()
            base = (wid * n_chunks + j) * CHUNK
            pltpu.sync_copy(rows_v, out_hbm.at[pl.ds(base, CHUNK)])
            return carry

        lax.fori_loop(0, n_chunks, body, 0)

    return grab


def kernel(x, weight):
    B, S = x.shape
    V, D = weight.shape
    total = B * S
    info = plsc.get_sparse_core_info()
    NW = info.num_cores * info.num_subcores
    n_chunks = total // (NW * CHUNK)
    xf = x.astype(jnp.int32).reshape(NW, n_chunks, CHUNK)
    out = _make_gather(V, D, total)(xf, weight)
    return out.reshape(B, S, D)


# speedup vs baseline: 8.0003x; 1.3832x over previous
import functools

import jax
import jax.numpy as jnp
from jax import lax
from jax.experimental import pallas as pl
from jax.experimental.pallas import tpu as pltpu
from jax.experimental.pallas import tpu_sc as plsc

CHUNK = 128
N_BUF = 4
AHEAD = 2


@functools.lru_cache(maxsize=None)
def _make_gather(V, D, B):
    info = plsc.get_sparse_core_info()
    NC, NS = info.num_cores, info.num_subcores
    NW = NC * NS
    assert B % (NW * CHUNK) == 0
    n_chunks = B // (NW * CHUNK)
    assert n_chunks > 2 * AHEAD and (n_chunks - AHEAD - N_BUF) % N_BUF == 0
    mesh = plsc.VectorSubcoreMesh(core_axis_name="c", subcore_axis_name="s")

    @functools.partial(
        pl.kernel,
        mesh=mesh,
        out_type=jax.ShapeDtypeStruct((B, D), jnp.float32),
        scratch_types=[
            pltpu.VMEM((n_chunks, CHUNK), jnp.int32),
        ]
        + [pltpu.VMEM((CHUNK, D), jnp.float32)] * N_BUF
        + [pltpu.SemaphoreType.DMA] * (2 * N_BUF),
    )
    def grab(x_hbm, w_hbm, out_hbm, idx_v, *rest):
        bufs = rest[:N_BUF]
        gsems = rest[N_BUF : 2 * N_BUF]
        osems = rest[2 * N_BUF :]
        wid = lax.axis_index("s") * NC + lax.axis_index("c")
        pltpu.sync_copy(x_hbm.at[wid], idx_v)
        out_base = wid * (n_chunks * CHUNK)

        def start_gather(j, slot):
            pltpu.async_copy(w_hbm.at[idx_v.at[j]], bufs[slot], gsems[slot])

        def wait_gather(j, slot):
            pltpu.make_async_copy(
                w_hbm.at[idx_v.at[j]], bufs[slot], gsems[slot]
            ).wait()

        def out_slice(j):
            return out_hbm.at[pl.ds(out_base + j * CHUNK, CHUNK)]

        def start_out(j, slot):
            pltpu.async_copy(bufs[slot], out_slice(j), osems[slot])

        def wait_out(j, slot):
            pltpu.make_async_copy(bufs[slot], out_slice(j), osems[slot]).wait()

        for j in range(AHEAD):
            start_gather(j, j % N_BUF)
        for j in range(AHEAD):
            start_gather(j + AHEAD, (j + AHEAD) % N_BUF)
            wait_gather(j, j % N_BUF)
            start_out(j, j % N_BUF)

        def body(i, carry):
            for b_off in range(N_BUF):
                j = AHEAD + N_BUF * i + b_off
                slot = (AHEAD + b_off) % N_BUF
                csl = (slot + AHEAD) % N_BUF
                wait_out(j - AHEAD, csl)
                start_gather(j + AHEAD, csl)
                wait_gather(j, slot)
                start_out(j, slot)
            return carry

        n_main = (n_chunks - AHEAD - N_BUF) // N_BUF
        lax.fori_loop(0, n_main, body, 0)

        for j in range(n_chunks - N_BUF, n_chunks):
            slot = j % N_BUF
            if j + AHEAD < n_chunks:
                csl = (slot + AHEAD) % N_BUF
                wait_out(j - AHEAD, csl)
                start_gather(j + AHEAD, csl)
            wait_gather(j, slot)
            start_out(j, slot)
        for j in range(n_chunks - N_BUF, n_chunks):
            wait_out(j, j % N_BUF)

    return grab


def kernel(x, weight):
    B, S = x.shape
    V, D = weight.shape
    total = B * S
    info = plsc.get_sparse_core_info()
    NW = info.num_cores * info.num_subcores
    n_chunks = total // (NW * CHUNK)
    xf = x.astype(jnp.int32).reshape(NW, n_chunks, CHUNK)
    out = _make_gather(V, D, total)(xf, weight)
    return out.reshape(B, S, D)


# speedup vs baseline: 8.0790x; 1.0098x over previous
import functools

import jax
import jax.numpy as jnp
from jax import lax
from jax.experimental import pallas as pl
from jax.experimental.pallas import tpu as pltpu
from jax.experimental.pallas import tpu_sc as plsc

CHUNK = 128
N_BUF = 6
AHEAD = 4


@functools.lru_cache(maxsize=None)
def _make_gather(V, D, B):
    info = plsc.get_sparse_core_info()
    NC, NS = info.num_cores, info.num_subcores
    NW = NC * NS
    assert B % (NW * CHUNK) == 0
    n_chunks = B // (NW * CHUNK)
    assert 0 < AHEAD < N_BUF and AHEAD % N_BUF != 0
    H = AHEAD + (n_chunks - AHEAD - N_BUF) % N_BUF
    n_main = (n_chunks - H - N_BUF) // N_BUF
    assert n_main >= 0 and H + AHEAD <= n_chunks and H >= N_BUF - AHEAD
    mesh = plsc.VectorSubcoreMesh(core_axis_name="c", subcore_axis_name="s")

    @functools.partial(
        pl.kernel,
        mesh=mesh,
        out_type=jax.ShapeDtypeStruct((B, D), jnp.float32),
        scratch_types=[
            pltpu.VMEM((n_chunks, CHUNK), jnp.int32),
        ]
        + [pltpu.VMEM((CHUNK, D), jnp.float32)] * N_BUF
        + [pltpu.SemaphoreType.DMA] * (2 * N_BUF),
    )
    def grab(x_hbm, w_hbm, out_hbm, idx_v, *rest):
        bufs = rest[:N_BUF]
        gsems = rest[N_BUF : 2 * N_BUF]
        osems = rest[2 * N_BUF :]
        wid = lax.axis_index("s") * NC + lax.axis_index("c")
        pltpu.sync_copy(x_hbm.at[wid], idx_v)
        out_base = wid * (n_chunks * CHUNK)

        def start_gather(j, slot):
            pltpu.async_copy(w_hbm.at[idx_v.at[j]], bufs[slot], gsems[slot])

        def wait_gather(j, slot):
            pltpu.make_async_copy(
                w_hbm.at[idx_v.at[j]], bufs[slot], gsems[slot]
            ).wait()

        def out_slice(j):
            return out_hbm.at[pl.ds(out_base + j * CHUNK, CHUNK)]

        def start_out(j, slot):
            pltpu.async_copy(bufs[slot], out_slice(j), osems[slot])

        def wait_out(j, slot):
            pltpu.make_async_copy(bufs[slot], out_slice(j), osems[slot]).wait()

        def retire(j, slot):
            wait_gather(j, slot)
            start_out(j, slot)

        for j in range(AHEAD):
            start_gather(j, j % N_BUF)
        for j in range(H):
            jg = j + AHEAD
            csl = jg % N_BUF
            if jg - N_BUF >= 0:
                wait_out(jg - N_BUF, csl)
            start_gather(jg, csl)
            retire(j, j % N_BUF)

        def body(i, carry):
            for b_off in range(N_BUF):
                j = H + N_BUF * i + b_off
                slot = (H + b_off) % N_BUF
                csl = (slot + AHEAD) % N_BUF
                wait_out(j + AHEAD - N_BUF, csl)
                start_gather(j + AHEAD, csl)
                retire(j, slot)
            return carry

        lax.fori_loop(0, n_main, body, 0)

        for j in range(n_chunks - N_BUF, n_chunks):
            jg = j + AHEAD
            if jg < n_chunks:
                csl = jg % N_BUF
                wait_out(jg - N_BUF, csl)
                start_gather(jg, csl)
            retire(j, j % N_BUF)
        for j in range(n_chunks - N_BUF, n_chunks):
            wait_out(j, j % N_BUF)

    return grab


def kernel(x, weight):
    B, S = x.shape
    V, D = weight.shape
    total = B * S
    info = plsc.get_sparse_core_info()
    NW = info.num_cores * info.num_subcores
    n_chunks = total // (NW * CHUNK)
    xf = x.astype(jnp.int32).reshape(NW, n_chunks, CHUNK)
    out = _make_gather(V, D, total)(xf, weight)
    return out.reshape(B, S, D)


# speedup vs baseline: 8.1272x; 1.0060x over previous
import functools

import jax
import jax.numpy as jnp
from jax import lax
from jax.experimental import pallas as pl
from jax.experimental.pallas import tpu as pltpu
from jax.experimental.pallas import tpu_sc as plsc

CHUNK = 128
N_BUF = 7
AHEAD = 5


@functools.lru_cache(maxsize=None)
def _make_gather(V, D, B):
    info = plsc.get_sparse_core_info()
    NC, NS = info.num_cores, info.num_subcores
    NW = NC * NS
    assert B % (NW * CHUNK) == 0
    n_chunks = B // (NW * CHUNK)
    assert 0 < AHEAD < N_BUF
    H = AHEAD + (n_chunks - AHEAD - N_BUF) % N_BUF
    n_main = (n_chunks - H - N_BUF) // N_BUF
    assert n_main >= 0 and H + AHEAD <= n_chunks and H >= N_BUF - AHEAD
    mesh = plsc.VectorSubcoreMesh(core_axis_name="c", subcore_axis_name="s")

    @functools.partial(
        pl.kernel,
        mesh=mesh,
        out_type=jax.ShapeDtypeStruct((B, D), jnp.float32),
        scratch_types=[
            pltpu.VMEM((n_chunks, CHUNK), jnp.int32),
        ]
        + [pltpu.VMEM((CHUNK, D), jnp.float32)] * N_BUF
        + [pltpu.SemaphoreType.DMA] * (2 * N_BUF),
    )
    def grab(x_hbm, w_hbm, out_hbm, idx_v, *rest):
        bufs = rest[:N_BUF]
        gsems = rest[N_BUF : 2 * N_BUF]
        osems = rest[2 * N_BUF :]
        wid = lax.axis_index("s") * NC + lax.axis_index("c")
        pltpu.sync_copy(x_hbm.at[wid], idx_v)
        out_base = wid * (n_chunks * CHUNK)

        def start_gather(j, slot):
            pltpu.async_copy(w_hbm.at[idx_v.at[j]], bufs[slot], gsems[slot])

        def wait_gather(j, slot):
            pltpu.make_async_copy(
                w_hbm.at[idx_v.at[j]], bufs[slot], gsems[slot]
            ).wait()

        def out_slice(j):
            return out_hbm.at[pl.ds(out_base + j * CHUNK, CHUNK)]

        def start_out(j, slot):
            pltpu.async_copy(bufs[slot], out_slice(j), osems[slot])

        def wait_out(j, slot):
            pltpu.make_async_copy(bufs[slot], out_slice(j), osems[slot]).wait()

        def retire(j, slot):
            wait_gather(j, slot)
            start_out(j, slot)

        for j in range(AHEAD):
            start_gather(j, j % N_BUF)
        for j in range(H):
            jg = j + AHEAD
            csl = jg % N_BUF
            if jg - N_BUF >= 0:
                wait_out(jg - N_BUF, csl)
            start_gather(jg, csl)
            retire(j, j % N_BUF)

        def body(i, carry):
            for b_off in range(N_BUF):
                j = H + N_BUF * i + b_off
                slot = (H + b_off) % N_BUF
                csl = (slot + AHEAD) % N_BUF
                wait_out(j + AHEAD - N_BUF, csl)
                start_gather(j + AHEAD, csl)
                retire(j, slot)
            return carry

        lax.fori_loop(0, n_main, body, 0)

        for j in range(n_chunks - N_BUF, n_chunks):
            jg = j + AHEAD
            if jg < n_chunks:
                csl = jg % N_BUF
                wait_out(jg - N_BUF, csl)
                start_gather(jg, csl)
            retire(j, j % N_BUF)
        for j in range(n_chunks - N_BUF, n_chunks):
            wait_out(j, j % N_BUF)

    return grab


def kernel(x, weight):
    B, S = x.shape
    V, D = weight.shape
    total = B * S
    info = plsc.get_sparse_core_info()
    NW = info.num_cores * info.num_subcores
    n_chunks = total // (NW * CHUNK)
    xf = x.astype(jnp.int32).reshape(NW, n_chunks, CHUNK)
    out = _make_gather(V, D, total)(xf, weight)
    return out.reshape(B, S, D)
